# baseline (speedup 1.0000x reference)
import functools

import jax
import jax.numpy as jnp
from jax import lax
from jax.experimental import pallas as pl
from jax.experimental.pallas import tpu as pltpu
from jax.experimental.pallas import tpu_sc as plsc

N_ATOMS = 100000
EMB = 128
CHUNK = 128
NC = 2
NS = 16
NW = NC * NS

_N_CHUNKS = -(-N_ATOMS // CHUNK)
_LAST_BASE = N_ATOMS - CHUNK
_PER_WORKER = 25
_PAIRS = (_PER_WORKER - 1) // 2


@functools.partial(
    pl.kernel,
    mesh=plsc.VectorSubcoreMesh(core_axis_name="c", subcore_axis_name="s"),
    out_type=jax.ShapeDtypeStruct((N_ATOMS, EMB), jnp.float32),
    scratch_types=[
        pltpu.VMEM((CHUNK,), jnp.int32),
        pltpu.VMEM((CHUNK,), jnp.int32),
        pltpu.VMEM((CHUNK, EMB), jnp.float32),
        pltpu.VMEM((CHUNK, EMB), jnp.float32),
        pltpu.SemaphoreType.DMA,
        pltpu.SemaphoreType.DMA,
    ],
)
def _emb_kernel(z_hbm, w_hbm, out_hbm, idx_a, idx_b, rows_a, rows_b,
                sem_a, sem_b):
    wid = lax.axis_index("s") * NC + lax.axis_index("c")

    def base_of(k):
        c = jnp.minimum(wid + k * NW, _N_CHUNKS - 1)
        return pl.multiple_of(jnp.minimum(c * CHUNK, _LAST_BASE), 8)

    def stage(k, idx_v, rows_v, sem):
        base = base_of(k)
        pltpu.sync_copy(z_hbm.at[pl.ds(base, CHUNK)], idx_v)
        for j in range(CHUNK // 16):
            sl = pl.ds(j * 16, 16)
            idx_v[sl] = idx_v[sl] - 1
        pltpu.async_copy(w_hbm.at[idx_v], rows_v, sem)

    def drain(k, idx_v, rows_v, sem):
        pltpu.make_async_copy(w_hbm.at[idx_v], rows_v, sem).wait()
        pltpu.sync_copy(rows_v, out_hbm.at[pl.ds(base_of(k), CHUNK)])

    stage(0, idx_a, rows_a, sem_a)

    def pair(p, _):
        k = 2 * p + 1
        stage(k, idx_b, rows_b, sem_b)
        drain(k - 1, idx_a, rows_a, sem_a)
        stage(k + 1, idx_a, rows_a, sem_a)
        drain(k, idx_b, rows_b, sem_b)
        return _

    lax.fori_loop(0, _PAIRS, pair, None)
    drain(_PER_WORKER - 1, idx_a, rows_a, sem_a)


def kernel(Z, W):
    return _emb_kernel(Z, W)
